# baseline (speedup 1.0000x reference)
import functools

import jax
import jax.numpy as jnp
from jax import lax
from jax.experimental import pallas as pl
from jax.experimental.pallas import tpu as pltpu
from jax.experimental.pallas import tpu_sc as plsc

N = 10000
D = 128
H = 128
G = 64

NC = 2
NS = 16
NW = NC * NS
CH = 128
E = 320000
K0 = 129
K1 = 30
E_PAD = NS * CH * (K0 + K1)
E_W0 = K0 * CH
E_W1 = K1 * CH
N_PAD = 10112
RPT = N_PAD // NS

def _seg_sum_body(h_hbm, src_hbm, dst_hbm, zeros_hbm, out_hbm,
                  srcv_a, srcv_b, srcv_c, dstv_a, dstv_b, dstv_c,
                  rows_a, rows_b, rows_c, acc,
                  gsa, gsb, gsc, isa, isb, isc):
    c = lax.axis_index("c")
    s = lax.axis_index("s")
    ebase = jnp.where(c == 0, s * E_W0, NS * E_W0 + s * E_W1)
    nchunk = jnp.where(c == 0, K0, K1)

    def idx_load(j, srcv, dstv, sem):
        pltpu.async_copy(src_hbm.at[pl.ds(ebase + j * CH, CH)], srcv, sem)
        pltpu.async_copy(dst_hbm.at[pl.ds(ebase + j * CH, CH)], dstv, sem)

    def idx_wait(srcv, dstv, sem):
        pltpu.make_async_copy(src_hbm.at[pl.ds(ebase, CH)], srcv, sem).wait()
        pltpu.make_async_copy(dst_hbm.at[pl.ds(ebase, CH)], dstv, sem).wait()

    def gather_wait(srcv, rows, sem):
        pltpu.make_async_copy(h_hbm.at[srcv], rows, sem).wait()

    pltpu.sync_copy(zeros_hbm, acc.at[pl.ds(s * RPT, RPT)])
    plsc.subcore_barrier()

    pltpu.sync_copy(src_hbm.at[pl.ds(ebase, CH)], srcv_a)
    pltpu.sync_copy(dst_hbm.at[pl.ds(ebase, CH)], dstv_a)
    pltpu.sync_copy(src_hbm.at[pl.ds(ebase + CH, CH)], srcv_b)
    pltpu.sync_copy(dst_hbm.at[pl.ds(ebase + CH, CH)], dstv_b)
    pltpu.async_copy(h_hbm.at[srcv_a], rows_a, gsa)
    pltpu.async_copy(h_hbm.at[srcv_b], rows_b, gsb)
    idx_load(2, srcv_c, dstv_c, isc)

    def stage(j, srcv, dstv, rows, gs, is_):
        gather_wait(srcv, rows, gs)
        pltpu.sync_copy(rows, acc.at[dstv], add=True)
        idx_load(j + 3, srcv, dstv, is_)
        idx_wait(srcv, dstv, is_)
        pltpu.async_copy(h_hbm.at[srcv], rows, gs)

    def body(i, carry):
        j = 3 * i
        idx_wait(srcv_c, dstv_c, isc)
        pltpu.async_copy(h_hbm.at[srcv_c], rows_c, gsc)
        stage(j, srcv_a, dstv_a, rows_a, gsa, isa)
        stage(j + 1, srcv_b, dstv_b, rows_b, gsb, isb)
        gather_wait(srcv_c, rows_c, gsc)
        pltpu.sync_copy(rows_c, acc.at[dstv_c], add=True)
        idx_load(j + 5, srcv_c, dstv_c, isc)
        return carry

    lax.fori_loop(0, nchunk // 3, body, 0)
    gather_wait(srcv_a, rows_a, gsa)
    gather_wait(srcv_b, rows_b, gsb)
    idx_wait(srcv_c, dstv_c, isc)
    plsc.subcore_barrier()
    pltpu.sync_copy(acc.at[pl.ds(s * RPT, RPT)],
                    out_hbm.at[pl.ds(c * N_PAD + s * RPT, RPT)])


@functools.cache
def _segment_sum_sc():
    mesh = plsc.VectorSubcoreMesh(core_axis_name="c", subcore_axis_name="s")
    return pl.kernel(
        _seg_sum_body,
        mesh=mesh,
        out_type=jax.ShapeDtypeStruct((2 * N_PAD, H), jnp.float32),
        scratch_types=[
            pltpu.VMEM((CH,), jnp.int32),
            pltpu.VMEM((CH,), jnp.int32),
            pltpu.VMEM((CH,), jnp.int32),
            pltpu.VMEM((CH,), jnp.int32),
            pltpu.VMEM((CH,), jnp.int32),
            pltpu.VMEM((CH,), jnp.int32),
            pltpu.VMEM((CH, H), jnp.float32),
            pltpu.VMEM((CH, H), jnp.float32),
            pltpu.VMEM((CH, H), jnp.float32),
            pltpu.VMEM_SHARED((N_PAD, H), jnp.float32),
            pltpu.SemaphoreType.DMA,
            pltpu.SemaphoreType.DMA,
            pltpu.SemaphoreType.DMA,
            pltpu.SemaphoreType.DMA,
            pltpu.SemaphoreType.DMA,
            pltpu.SemaphoreType.DMA,
        ],
    )


BLK = 1000


def _mlp_body(eps_ref, x_ref, a0_ref, a1_ref, w1_ref, b1_ref, w2_ref,
              b2_ref, o_ref, *, relu_out):
    h = (1.0 + eps_ref[0]) * x_ref[...] + a0_ref[...] + a1_ref[...]
    h = jnp.dot(h, w1_ref[...], preferred_element_type=jnp.float32)
    h = jnp.maximum(h + b1_ref[...], 0.0)
    h = jnp.dot(h, w2_ref[...], preferred_element_type=jnp.float32)
    h = h + b2_ref[...]
    if relu_out:
        h = jnp.maximum(h, 0.0)
    o_ref[...] = h


def _mlp_layer(x, a0, a1, w1, b1, w2, b2, eps, relu_out):
    grid = N // BLK
    return pl.pallas_call(
        functools.partial(_mlp_body, relu_out=relu_out),
        grid=(grid,),
        in_specs=[
            pl.BlockSpec(memory_space=pltpu.SMEM),
            pl.BlockSpec((BLK, H), lambda i: (i, 0)),
            pl.BlockSpec((BLK, H), lambda i: (i, 0)),
            pl.BlockSpec((BLK, H), lambda i: (i, 0)),
            pl.BlockSpec((H, H), lambda i: (0, 0)),
            pl.BlockSpec((1, H), lambda i: (0, 0)),
            pl.BlockSpec((H, H), lambda i: (0, 0)),
            pl.BlockSpec((1, H), lambda i: (0, 0)),
        ],
        out_specs=pl.BlockSpec((BLK, H), lambda i: (i, 0)),
        out_shape=jax.ShapeDtypeStruct((N, H), jnp.float32),
    )(eps.reshape(1), x, a0, a1, w1, b1.reshape(1, H), w2, b2.reshape(1, H))


def _mlp_pool_body(eps_ref, batch_ref, x_ref, a0_ref, a1_ref, w1_ref,
                   b1_ref, w2_ref, b2_ref, fcw_ref, fcb_ref, o_ref,
                   sums_acc, cnt_acc):
    i = pl.program_id(0)

    @pl.when(i == 0)
    def _init():
        sums_acc[...] = jnp.zeros_like(sums_acc)
        cnt_acc[...] = jnp.zeros_like(cnt_acc)

    h = (1.0 + eps_ref[0]) * x_ref[...] + a0_ref[...] + a1_ref[...]
    h = jnp.dot(h, w1_ref[...], preferred_element_type=jnp.float32)
    h = jnp.maximum(h + b1_ref[...], 0.0)
    h = jnp.dot(h, w2_ref[...], preferred_element_type=jnp.float32)
    h = h + b2_ref[...]

    b = batch_ref[0, 0, :]
    onehot_t = (lax.broadcasted_iota(jnp.int32, (G, BLK), 0)
                == b[None, :]).astype(jnp.float32)
    sums_acc[...] += lax.dot_general(
        onehot_t, h, (((1,), (0,)), ((), ())),
        preferred_element_type=jnp.float32)
    cnt_acc[...] += jnp.broadcast_to(
        jnp.sum(onehot_t, axis=1, keepdims=True), (G, H))

    @pl.when(i == pl.num_programs(0) - 1)
    def _final():
        pooled = sums_acc[...] / jnp.maximum(cnt_acc[...], 1.0)
        out = jnp.dot(pooled, fcw_ref[...],
                      preferred_element_type=jnp.float32)
        o_ref[...] = out + fcb_ref[0]


def _mlp_pool_layer(x, a0, a1, batch_r, w1, b1, w2, b2, eps, fcw, fcb):
    grid = N // BLK
    out = pl.pallas_call(
        _mlp_pool_body,
        grid=(grid,),
        in_specs=[
            pl.BlockSpec(memory_space=pltpu.SMEM),
            pl.BlockSpec((1, 1, BLK), lambda i: (i, 0, 0)),
            pl.BlockSpec((BLK, H), lambda i: (i, 0)),
            pl.BlockSpec((BLK, H), lambda i: (i, 0)),
            pl.BlockSpec((BLK, H), lambda i: (i, 0)),
            pl.BlockSpec((H, H), lambda i: (0, 0)),
            pl.BlockSpec((1, H), lambda i: (0, 0)),
            pl.BlockSpec((H, H), lambda i: (0, 0)),
            pl.BlockSpec((1, H), lambda i: (0, 0)),
            pl.BlockSpec((H, 1), lambda i: (0, 0)),
            pl.BlockSpec(memory_space=pltpu.SMEM),
        ],
        out_specs=pl.BlockSpec((G, 1), lambda i: (0, 0)),
        out_shape=jax.ShapeDtypeStruct((G, 1), jnp.float32),
        scratch_shapes=[
            pltpu.VMEM((G, H), jnp.float32),
            pltpu.VMEM((G, H), jnp.float32),
        ],
    )(eps.reshape(1), batch_r, x, a0, a1, w1, b1.reshape(1, H), w2,
      b2.reshape(1, H), fcw, fcb.reshape(1))
    return out[:, 0]


def kernel(x, edge_index, batch, W1_0, b1_0, W2_0, b2_0, eps_0, W1_1,
           b1_1, W2_1, b2_1, eps_1, W1_2, b1_2, W2_2, b2_2, eps_2, fcW,
           fcb):
    pad = E_PAD - E + 3 * CH
    src = jnp.concatenate([edge_index[0], jnp.zeros((pad,), jnp.int32)])
    dst = jnp.concatenate(
        [edge_index[1], jnp.full((pad,), N, jnp.int32)])
    zeros_tile = jnp.zeros((RPT, H), jnp.float32)
    batch_r = batch.reshape(N // BLK, 1, BLK)

    params = [(W1_0, b1_0, W2_0, b2_0, eps_0),
              (W1_1, b1_1, W2_1, b2_1, eps_1),
              (W1_2, b1_2, W2_2, b2_2, eps_2)]
    h = x
    for i, (w1, b1, w2, b2, eps) in enumerate(params):
        parts = _segment_sum_sc()(h, src, dst, zeros_tile)
        a0 = parts[0:N]
        a1 = parts[N_PAD:N_PAD + N]
        if i < 2:
            h = _mlp_layer(h, a0, a1, w1, b1, w2, b2, eps, relu_out=True)
        else:
            return _mlp_pool_layer(h, a0, a1, batch_r, w1, b1, w2, b2,
                                   eps, fcW, fcb)


# speedup vs baseline: 1.5127x; 1.5127x over previous
import functools

import jax
import jax.numpy as jnp
from jax import lax
from jax.experimental import pallas as pl
from jax.experimental.pallas import tpu as pltpu
from jax.experimental.pallas import tpu_sc as plsc

N = 10000
D = 128
H = 128
G = 64

NC = 2
NS = 16
NW = NC * NS
CH = 128
E = 320000
K0 = 128
K1 = 30
E_PAD = NS * CH * (K0 + K1)
E_W0 = K0 * CH
E_W1 = K1 * CH
N_PAD = 10112
RPT = N_PAD // NS

def _seg_sum_body(h_hbm, src_hbm, dst_hbm, zeros_hbm, out_hbm,
                  srcv_a, srcv_b, dstv_a, dstv_b, rows_a, rows_b, acc,
                  gsa, gsb, isa, isb):
    c = lax.axis_index("c")
    s = lax.axis_index("s")
    ebase = jnp.where(c == 0, s * E_W0, NS * E_W0 + s * E_W1)
    nchunk = jnp.where(c == 0, K0, K1)

    def idx_load(j, srcv, dstv, sem):
        pltpu.async_copy(src_hbm.at[pl.ds(ebase + j * CH, CH)], srcv, sem)
        pltpu.async_copy(dst_hbm.at[pl.ds(ebase + j * CH, CH)], dstv, sem)

    def idx_wait(srcv, dstv, sem):
        pltpu.make_async_copy(src_hbm.at[pl.ds(ebase, CH)], srcv, sem).wait()
        pltpu.make_async_copy(dst_hbm.at[pl.ds(ebase, CH)], dstv, sem).wait()

    def gather_wait(srcv, rows, sem):
        pltpu.make_async_copy(h_hbm.at[srcv], rows, sem).wait()

    pltpu.sync_copy(zeros_hbm, acc.at[pl.ds(s * RPT, RPT)])
    plsc.subcore_barrier()

    pltpu.sync_copy(src_hbm.at[pl.ds(ebase, CH)], srcv_a)
    pltpu.sync_copy(dst_hbm.at[pl.ds(ebase, CH)], dstv_a)
    pltpu.async_copy(h_hbm.at[srcv_a], rows_a, gsa)
    idx_load(1, srcv_b, dstv_b, isb)

    def body(i, carry):
        j = 2 * i
        idx_wait(srcv_b, dstv_b, isb)
        pltpu.async_copy(h_hbm.at[srcv_b], rows_b, gsb)
        gather_wait(srcv_a, rows_a, gsa)
        pltpu.sync_copy(rows_a, acc.at[dstv_a], add=True)
        idx_load(j + 2, srcv_a, dstv_a, isa)
        idx_wait(srcv_a, dstv_a, isa)
        pltpu.async_copy(h_hbm.at[srcv_a], rows_a, gsa)
        gather_wait(srcv_b, rows_b, gsb)
        pltpu.sync_copy(rows_b, acc.at[dstv_b], add=True)
        idx_load(j + 3, srcv_b, dstv_b, isb)
        return carry

    lax.fori_loop(0, nchunk // 2, body, 0)
    gather_wait(srcv_a, rows_a, gsa)
    idx_wait(srcv_b, dstv_b, isb)
    plsc.subcore_barrier()
    pltpu.sync_copy(acc.at[pl.ds(s * RPT, RPT)],
                    out_hbm.at[pl.ds(c * N_PAD + s * RPT, RPT)])


@functools.cache
def _segment_sum_sc():
    mesh = plsc.VectorSubcoreMesh(core_axis_name="c", subcore_axis_name="s")
    return pl.kernel(
        _seg_sum_body,
        mesh=mesh,
        out_type=jax.ShapeDtypeStruct((2 * N_PAD, H), jnp.float32),
        scratch_types=[
            pltpu.VMEM((CH,), jnp.int32),
            pltpu.VMEM((CH,), jnp.int32),
            pltpu.VMEM((CH,), jnp.int32),
            pltpu.VMEM((CH,), jnp.int32),
            pltpu.VMEM((CH, H), jnp.float32),
            pltpu.VMEM((CH, H), jnp.float32),
            pltpu.VMEM_SHARED((N_PAD, H), jnp.float32),
            pltpu.SemaphoreType.DMA,
            pltpu.SemaphoreType.DMA,
            pltpu.SemaphoreType.DMA,
            pltpu.SemaphoreType.DMA,
        ],
    )


BLK = 1000


def _mlp_body(eps_ref, x_ref, a0_ref, a1_ref, w1_ref, b1_ref, w2_ref,
              b2_ref, o_ref, *, relu_out):
    h = (1.0 + eps_ref[0]) * x_ref[...] + a0_ref[...] + a1_ref[...]
    h = jnp.dot(h, w1_ref[...], preferred_element_type=jnp.float32)
    h = jnp.maximum(h + b1_ref[...], 0.0)
    h = jnp.dot(h, w2_ref[...], preferred_element_type=jnp.float32)
    h = h + b2_ref[...]
    if relu_out:
        h = jnp.maximum(h, 0.0)
    o_ref[...] = h


def _mlp_layer(x, a0, a1, w1, b1, w2, b2, eps, relu_out):
    grid = N // BLK
    return pl.pallas_call(
        functools.partial(_mlp_body, relu_out=relu_out),
        grid=(grid,),
        in_specs=[
            pl.BlockSpec(memory_space=pltpu.SMEM),
            pl.BlockSpec((BLK, H), lambda i: (i, 0)),
            pl.BlockSpec((BLK, H), lambda i: (i, 0)),
            pl.BlockSpec((BLK, H), lambda i: (i, 0)),
            pl.BlockSpec((H, H), lambda i: (0, 0)),
            pl.BlockSpec((1, H), lambda i: (0, 0)),
            pl.BlockSpec((H, H), lambda i: (0, 0)),
            pl.BlockSpec((1, H), lambda i: (0, 0)),
        ],
        out_specs=pl.BlockSpec((BLK, H), lambda i: (i, 0)),
        out_shape=jax.ShapeDtypeStruct((N, H), jnp.float32),
    )(eps.reshape(1), x, a0, a1, w1, b1.reshape(1, H), w2, b2.reshape(1, H))


def _mlp_pool_body(eps_ref, batch_ref, x_ref, a0_ref, a1_ref, w1_ref,
                   b1_ref, w2_ref, b2_ref, fcw_ref, fcb_ref, o_ref,
                   sums_acc, cnt_acc):
    i = pl.program_id(0)

    @pl.when(i == 0)
    def _init():
        sums_acc[...] = jnp.zeros_like(sums_acc)
        cnt_acc[...] = jnp.zeros_like(cnt_acc)

    h = (1.0 + eps_ref[0]) * x_ref[...] + a0_ref[...] + a1_ref[...]
    h = jnp.dot(h, w1_ref[...], preferred_element_type=jnp.float32)
    h = jnp.maximum(h + b1_ref[...], 0.0)
    h = jnp.dot(h, w2_ref[...], preferred_element_type=jnp.float32)
    h = h + b2_ref[...]

    b = batch_ref[0, 0, :]
    onehot_t = (lax.broadcasted_iota(jnp.int32, (G, BLK), 0)
                == b[None, :]).astype(jnp.float32)
    sums_acc[...] += lax.dot_general(
        onehot_t, h, (((1,), (0,)), ((), ())),
        preferred_element_type=jnp.float32)
    cnt_acc[...] += jnp.broadcast_to(
        jnp.sum(onehot_t, axis=1, keepdims=True), (G, H))

    @pl.when(i == pl.num_programs(0) - 1)
    def _final():
        pooled = sums_acc[...] / jnp.maximum(cnt_acc[...], 1.0)
        out = jnp.dot(pooled, fcw_ref[...],
                      preferred_element_type=jnp.float32)
        o_ref[...] = out + fcb_ref[0]


def _mlp_pool_layer(x, a0, a1, batch_r, w1, b1, w2, b2, eps, fcw, fcb):
    grid = N // BLK
    out = pl.pallas_call(
        _mlp_pool_body,
        grid=(grid,),
        in_specs=[
            pl.BlockSpec(memory_space=pltpu.SMEM),
            pl.BlockSpec((1, 1, BLK), lambda i: (i, 0, 0)),
            pl.BlockSpec((BLK, H), lambda i: (i, 0)),
            pl.BlockSpec((BLK, H), lambda i: (i, 0)),
            pl.BlockSpec((BLK, H), lambda i: (i, 0)),
            pl.BlockSpec((H, H), lambda i: (0, 0)),
            pl.BlockSpec((1, H), lambda i: (0, 0)),
            pl.BlockSpec((H, H), lambda i: (0, 0)),
            pl.BlockSpec((1, H), lambda i: (0, 0)),
            pl.BlockSpec((H, 1), lambda i: (0, 0)),
            pl.BlockSpec(memory_space=pltpu.SMEM),
        ],
        out_specs=pl.BlockSpec((G, 1), lambda i: (0, 0)),
        out_shape=jax.ShapeDtypeStruct((G, 1), jnp.float32),
        scratch_shapes=[
            pltpu.VMEM((G, H), jnp.float32),
            pltpu.VMEM((G, H), jnp.float32),
        ],
    )(eps.reshape(1), batch_r, x, a0, a1, w1, b1.reshape(1, H), w2,
      b2.reshape(1, H), fcw, fcb.reshape(1))
    return out[:, 0]


def kernel(x, edge_index, batch, W1_0, b1_0, W2_0, b2_0, eps_0, W1_1,
           b1_1, W2_1, b2_1, eps_1, W1_2, b1_2, W2_2, b2_2, eps_2, fcW,
           fcb):
    pad = E_PAD - E + 2 * CH
    src = jnp.concatenate([edge_index[0], jnp.zeros((pad,), jnp.int32)])
    dst = jnp.concatenate(
        [edge_index[1], jnp.full((pad,), N, jnp.int32)])
    zeros_tile = jnp.zeros((RPT, H), jnp.float32)
    batch_r = batch.reshape(N // BLK, 1, BLK)

    params = [(W1_0, b1_0, W2_0, b2_0, eps_0),
              (W1_1, b1_1, W2_1, b2_1, eps_1),
              (W1_2, b1_2, W2_2, b2_2, eps_2)]
    h = x
    for i, (w1, b1, w2, b2, eps) in enumerate(params):
        parts = _segment_sum_sc()(h, src, dst, zeros_tile)
        a0 = parts[0:N]
        a1 = parts[N_PAD:N_PAD + N]
        if i < 2:
            h = _mlp_layer(h, a0, a1, w1, b1, w2, b2, eps, relu_out=True)
        else:
            return _mlp_pool_layer(h, a0, a1, batch_r, w1, b1, w2, b2,
                                   eps, fcW, fcb)
